# Initial kernel scaffold; baseline (speedup 1.0000x reference)
import jax
import jax.numpy as jnp
from jax.experimental import pallas as pl


def kernel(input, embed, bi):
    raise NotImplementedError("write your pallas kernel here")


# speedup vs baseline: 1.1552x; 1.1552x over previous
import functools

import jax
import jax.numpy as jnp
from jax import lax
from jax.experimental import pallas as pl
from jax.experimental.pallas import tpu as pltpu
from jax.experimental.pallas import tpu_sc as plsc

DIM = 64
N_EMBED = 1024
POS_DIM = 16
POS_EMBED = 128
TOKENS = 16384
BLOCK = 1024
GRID = TOKENS // BLOCK


def _tc_body(x_ref, emb_ref, gate_ref, ind_ref, diff_ref, cb_ref):
    b = pl.program_id(0)
    x = x_ref[...]
    emb = emb_ref[...]

    row = lax.broadcasted_iota(jnp.int32, (DIM, N_EMBED), 0)
    col = lax.broadcasted_iota(jnp.int32, (DIM, N_EMBED), 1)
    mask = ((row < DIM - POS_DIM) == (col < N_EMBED - POS_EMBED)).astype(
        jnp.float32)
    emb_masked = emb * mask
    cb = emb_masked * gate_ref[0, 0]

    @pl.when(b == 0)
    def _():
        cb_ref[...] = emb_masked

    xcb = jnp.dot(x, cb, preferred_element_type=jnp.float32)
    fnorm = jnp.sum(x * x, axis=1, keepdims=True)
    cnorm = jnp.sum(cb * cb, axis=0, keepdims=True)
    dist = fnorm - 2.0 * xcb + cnorm

    mind = jnp.min(dist, axis=1, keepdims=True)
    cidx = lax.broadcasted_iota(jnp.int32, (BLOCK, N_EMBED), 1)
    ind = jnp.min(jnp.where(dist == mind, cidx, N_EMBED), axis=1,
                  keepdims=True)
    ind_ref[...] = ind
    diff_ref[...] = (jnp.sum(mind) / jnp.float32(BLOCK * DIM)).reshape(1, 1, 1)


def _tc_stage(flat, embed, gate):
    return pl.pallas_call(
        _tc_body,
        grid=(GRID,),
        in_specs=[
            pl.BlockSpec((BLOCK, DIM), lambda b: (b, 0)),
            pl.BlockSpec((DIM, N_EMBED), lambda b: (0, 0)),
            pl.BlockSpec((1, 1), lambda b: (0, 0)),
        ],
        out_specs=[
            pl.BlockSpec((BLOCK, 1), lambda b: (b, 0)),
            pl.BlockSpec((1, 1, 1), lambda b: (b, 0, 0)),
            pl.BlockSpec((DIM, N_EMBED), lambda b: (0, 0)),
        ],
        out_shape=[
            jax.ShapeDtypeStruct((TOKENS, 1), jnp.int32),
            jax.ShapeDtypeStruct((GRID, 1, 1), jnp.float32),
            jax.ShapeDtypeStruct((DIM, N_EMBED), jnp.float32),
        ],
    )(flat, embed, gate)


_CHUNK = 128


def _sc_gather(table, idx2d):
    info = plsc.get_sparse_core_info()
    _NC, _NS = info.num_cores, info.num_subcores
    _NW = _NC * _NS
    _BPW = TOKENS // _NW
    _NCHUNK = _BPW // _CHUNK
    mesh = plsc.VectorSubcoreMesh(core_axis_name="c", subcore_axis_name="s")

    @functools.partial(
        pl.kernel,
        mesh=mesh,
        compiler_params=pltpu.CompilerParams(use_tc_tiling_on_sc=False),
        out_type=jax.ShapeDtypeStruct((TOKENS, DIM), jnp.float32),
        scratch_types=[
            pltpu.VMEM((_NCHUNK, _CHUNK), jnp.int32),
            pltpu.VMEM((_BPW, DIM), jnp.float32),
            pltpu.SemaphoreType.DMA,
        ],
    )
    def k(table_hbm, idx_hbm, out_hbm, idx_v, rows_v, sem):
        wid = lax.axis_index("s") * _NC + lax.axis_index("c")
        pltpu.sync_copy(idx_hbm.at[pl.ds(wid * _NCHUNK, _NCHUNK), :], idx_v)
        copies = [
            pltpu.async_copy(
                table_hbm.at[idx_v.at[j]],
                rows_v.at[pl.ds(j * _CHUNK, _CHUNK), :],
                sem,
            )
            for j in range(_NCHUNK)
        ]
        for c in copies:
            c.wait()
        pltpu.sync_copy(rows_v, out_hbm.at[pl.ds(wid * _BPW, _BPW), :])

    return k(table, idx2d)


def kernel(input, embed, bi):
    flat = input.reshape(TOKENS, DIM)
    gate = (jnp.asarray(bi) == 1).astype(jnp.float32).reshape(1, 1)
    ind, diff, cb = _tc_stage(flat, embed, gate)
    q = _sc_gather(cb.T, ind.reshape(TOKENS // _CHUNK, _CHUNK))
    q = (q * gate[0, 0]).reshape(input.shape)
    quantize = input + lax.stop_gradient(q - input)
    embed_ind = ind.reshape(input.shape[:-1])
    return quantize, diff.reshape(GRID), embed_ind, cb
